# baseline (speedup 1.0000x reference)
import functools

import jax
import jax.numpy as jnp
from jax import lax
from jax.experimental import pallas as pl
from jax.experimental.pallas import tpu as pltpu
from jax.experimental.pallas import tpu_sc as plsc

N = 10000
E = 320000
D = 128
L = 6

NC = 2
NS = 16
NW = NC * NS
NPAD = 10240
NT = NPAD // NS
EPAD = NW * NPAD
ROWS_W = 80
CHUNK = 128
OROW = NPAD // NS

_mesh = plsc.VectorSubcoreMesh(core_axis_name="c", subcore_axis_name="s")


def _dot_nt(a, b):
    return lax.dot_general(a, b, (((1,), (1,)), ((), ())),
                           preferred_element_type=jnp.float32)


_EB = 6400
_GB = E // _EB


def _p1_body(We_ref, ae_ref, ea_ref, ecC_ref, ecm_ref, acc_ref):
    i = pl.program_id(0)
    ecC_ref[...] = jnp.zeros((_EB, 8), jnp.float32)

    @pl.when(i == 0)
    def _():
        acc_ref[...] = jnp.zeros((8, D), jnp.float32)

    for l in range(L):
        he = jnp.dot(ea_ref[...], We_ref[l],
                     preferred_element_type=jnp.float32)
        ec = jnp.sum(he * ae_ref[l][None, :], axis=1, keepdims=True)
        ecC_ref[:, l:l + 1] = ec
        m = jnp.max(ec)
        acc_ref[l, :] = jnp.maximum(acc_ref[l, :], jnp.full((D,), m))

    @pl.when(i == _GB - 1)
    def _():
        ecm_ref[...] = acc_ref[...]


def _p1(We, a_edge, edge_attr):
    return pl.pallas_call(
        _p1_body,
        grid=(_GB,),
        in_specs=[
            pl.BlockSpec((L, D, D), lambda i: (0, 0, 0)),
            pl.BlockSpec((L, D), lambda i: (0, 0)),
            pl.BlockSpec((_EB, D), lambda i: (i, 0)),
        ],
        out_specs=[
            pl.BlockSpec((_EB, 8), lambda i: (i, 0)),
            pl.BlockSpec((8, D), lambda i: (0, 0)),
        ],
        out_shape=[
            jax.ShapeDtypeStruct((E, 8), jnp.float32),
            jax.ShapeDtypeStruct((8, D), jnp.float32),
        ],
        scratch_shapes=[pltpu.VMEM((8, D), jnp.float32)],
    )(We, a_edge, edge_attr)


def _p2_body(h_ref, w_ref, asrc_ref, adst_ref, hw_ref, s_ref, d_ref):
    h = h_ref[...]
    hw = jnp.dot(h, w_ref[...], preferred_element_type=jnp.float32)
    hw_ref[...] = hw
    s_ref[...] = jnp.sum(hw * asrc_ref[...], axis=1, keepdims=True)
    d_ref[...] = jnp.sum(hw * adst_ref[...], axis=1, keepdims=True)


def _p2(h, w_l, asrc_row, adst_row):
    return pl.pallas_call(
        _p2_body,
        out_shape=[
            jax.ShapeDtypeStruct((N, D), jnp.float32),
            jax.ShapeDtypeStruct((N, 1), jnp.float32),
            jax.ShapeDtypeStruct((N, 1), jnp.float32),
        ],
    )(h, w_l, asrc_row, adst_row)


def _p2b_body(den_ref, rec_ref):
    rec_ref[...] = 1.0 / (den_ref[0, :][None, :] + den_ref[1, :][None, :]
                          + 1e-16)


def _p2b(den):
    return pl.pallas_call(
        _p2b_body,
        out_shape=jax.ShapeDtypeStruct((1, NPAD), jnp.float32),
    )(den)


def _p3_body(pp_ref, bias_ref, gamma_ref, beta_ref, h_ref):
    out = pp_ref[0, :N, :] + pp_ref[1, :N, :] + bias_ref[...]
    mu = jnp.mean(out, axis=0, keepdims=True)
    cen = out - mu
    var = jnp.mean(cen * cen, axis=0, keepdims=True)
    h_ref[...] = cen * lax.rsqrt(var + 1e-5) * gamma_ref[...] + beta_ref[...]


def _p3(pp, bias_row, gamma_row, beta_row):
    return pl.pallas_call(
        _p3_body,
        out_shape=jax.ShapeDtypeStruct((N, D), jnp.float32),
    )(pp, bias_row, gamma_row, beta_row)


@functools.partial(
    pl.kernel,
    out_type=[
        jax.ShapeDtypeStruct((NW * ROWS_W, CHUNK), jnp.float32),
        jax.ShapeDtypeStruct((NC, NPAD), jnp.float32),
    ],
    mesh=_mesh,
    compiler_params=pltpu.CompilerParams(needs_layout_passes=False),
    scratch_types=[
        pltpu.VMEM((NPAD,), jnp.float32),
        pltpu.VMEM((NPAD,), jnp.float32),
        pltpu.VMEM((ROWS_W, CHUNK), jnp.int32),
        pltpu.VMEM((ROWS_W, CHUNK), jnp.int32),
        pltpu.VMEM((ROWS_W, CHUNK), jnp.float32),
        pltpu.VMEM((ROWS_W, CHUNK), jnp.float32),
        pltpu.VMEM((16,), jnp.float32),
        pltpu.VMEM((128,), jnp.float32),
        pltpu.VMEM((NT,), jnp.float32),
        pltpu.VMEM_SHARED((NPAD,), jnp.float32),
        pltpu.SemaphoreType.DMA,
    ],
)
def _sc_attn(sd_hbm, ec_hbm, ecm_hbm, src_hbm, dst_hbm,
             ex_hbm, den_hbm,
             s_v, d_v, src_v, dst_v, ec_v, ex_v, ecm_v, tmp_v, z_v, den_sh,
             dsem):
    c = lax.axis_index("c")
    sid = lax.axis_index("s")
    wid = c * NS + sid
    rbase = wid * ROWS_W

    pltpu.sync_copy(sd_hbm.at[0], s_v)
    pltpu.sync_copy(sd_hbm.at[1], d_v)
    pltpu.sync_copy(src_hbm.at[pl.ds(rbase, ROWS_W)], src_v)
    pltpu.sync_copy(dst_hbm.at[pl.ds(rbase, ROWS_W)], dst_v)
    pltpu.sync_copy(ec_hbm.at[pl.ds(rbase, ROWS_W)], ec_v)
    pltpu.sync_copy(ecm_hbm, ecm_v)

    def mx_body(j, carry):
        ms, md = carry
        ms = jnp.maximum(ms, s_v[pl.ds(j * 16, 16)])
        md = jnp.maximum(md, d_v[pl.ds(j * 16, 16)])
        return ms, md

    init = jnp.zeros((16,), jnp.float32)
    ms, md = lax.fori_loop(0, NPAD // 16, mx_body, (init, init))
    lanes = lax.iota(jnp.int32, 16)

    def lane_max(v):
        for sh in (8, 4, 2, 1):
            tmp_v[pl.ds(0, 16)] = v
            g = plsc.load_gather(tmp_v, [lanes ^ sh])
            v = jnp.maximum(v, g)
        return v

    cbound = lane_max(ms) + lane_max(md) + ecm_v[...]
    cvec = jnp.maximum(cbound, 0.0)

    def row_body(r, _):
        for q in range(CHUNK // 16):
            col = q * 16
            si = src_v[r, pl.ds(col, 16)]
            di = dst_v[r, pl.ds(col, 16)]
            sg = plsc.load_gather(s_v, [si])
            dg = plsc.load_gather(d_v, [di])
            lg = sg + dg + ec_v[r, pl.ds(col, 16)]
            lg = jnp.where(lg > 0, lg, lg * 0.2) - cvec
            ex = jnp.exp(lg)
            gid = wid * (ROWS_W * CHUNK) + r * CHUNK + col + lanes
            ex_v[r, pl.ds(col, 16)] = jnp.where(gid < E, ex, 0.0)
        return 0

    lax.fori_loop(0, ROWS_W, row_body, 0)

    pltpu.sync_copy(ex_v, ex_hbm.at[pl.ds(rbase, ROWS_W)])

    def z_body(j, _):
        z_v[pl.ds(j * 16, 16)] = jnp.zeros((16,), jnp.float32)
        return 0

    lax.fori_loop(0, NT // 16, z_body, 0)
    pltpu.sync_copy(z_v, den_sh.at[pl.ds(sid * NT, NT)])
    plsc.subcore_barrier()

    def sc_body(r, _):
        pltpu.async_copy(ex_v.at[r], den_sh.at[dst_v.at[r]], dsem, add=True)
        return 0

    lax.fori_loop(0, ROWS_W, sc_body, 0)

    def dr_body(r, _):
        pltpu.make_async_copy(ex_v.at[0], den_sh.at[dst_v.at[0]],
                              dsem).wait()
        return 0

    lax.fori_loop(0, ROWS_W, dr_body, 0)
    plsc.subcore_barrier()

    pltpu.sync_copy(den_sh.at[pl.ds(sid * NT, NT)],
                    den_hbm.at[c, pl.ds(sid * NT, NT)])


STAGES = 10
SROWS = ROWS_W // STAGES


@functools.partial(
    pl.kernel,
    out_type=jax.ShapeDtypeStruct((NC, NPAD, D), jnp.float32),
    mesh=_mesh,
    compiler_params=pltpu.CompilerParams(needs_layout_passes=False),
    scratch_types=[
        pltpu.VMEM((NPAD,), jnp.float32),
        pltpu.VMEM((SROWS, CHUNK), jnp.int32),
        pltpu.VMEM((SROWS, CHUNK), jnp.int32),
        pltpu.VMEM((SROWS, CHUNK), jnp.float32),
        pltpu.VMEM((CHUNK, D), jnp.float32),
        pltpu.VMEM((CHUNK, D), jnp.float32),
        pltpu.VMEM_SHARED((NPAD, D), jnp.float32),
        pltpu.SemaphoreType.DMA,
        pltpu.SemaphoreType.DMA,
        pltpu.SemaphoreType.DMA,
        pltpu.SemaphoreType.DMA,
    ],
)
def _sc_aggr(hw_hbm, src_hbm, dst_hbm, ex_hbm, rec_hbm,
             out_hbm,
             rec_v, src_v, dst_v, ex_v, rows_a, rows_b, out_sh,
             gsa, gsb, ssa, ssb):
    c = lax.axis_index("c")
    sid = lax.axis_index("s")
    wid = c * NS + sid
    rbase = wid * ROWS_W

    pltpu.sync_copy(rec_hbm.at[0], rec_v)

    def zr_body(r, _):
        def zq_body(q, _):
            rows_a[r, pl.ds(q * 16, 16)] = jnp.zeros((16,), jnp.float32)
            return 0

        lax.fori_loop(0, D // 16, zq_body, 0)
        return 0

    lax.fori_loop(0, CHUNK, zr_body, 0)
    obase = sid * OROW
    for k in range(OROW // CHUNK):
        pltpu.sync_copy(rows_a, out_sh.at[pl.ds(obase + k * CHUNK, CHUNK)])
    plsc.subcore_barrier()

    def scale(rows_ref, k):
        def r_body(r, _):
            a16 = plsc.load_gather(
                ex_v, [jnp.full((16,), k, jnp.int32),
                       jnp.full((16,), r, jnp.int32)])
            for b in range(D // 16):
                sl = pl.ds(b * 16, 16)
                rows_ref[r, sl] = rows_ref[r, sl] * a16
            return 0

        lax.fori_loop(0, CHUNK, r_body, 0)

    def st_body(st, _):
        sbase = rbase + st * SROWS
        pltpu.sync_copy(src_hbm.at[pl.ds(sbase, SROWS)], src_v)
        pltpu.sync_copy(dst_hbm.at[pl.ds(sbase, SROWS)], dst_v)
        pltpu.sync_copy(ex_hbm.at[pl.ds(sbase, SROWS)], ex_v)

        def al_body(r, _):
            for q in range(CHUNK // 16):
                sl = pl.ds(q * 16, 16)
                di = dst_v[r, sl]
                ex_v[r, sl] = ex_v[r, sl] * plsc.load_gather(rec_v, [di])
            return 0

        lax.fori_loop(0, SROWS, al_body, 0)

        pltpu.async_copy(hw_hbm.at[src_v.at[0]], rows_a, gsa)

        npair = SROWS // 2

        def pair_body(t, _):
            k0 = 2 * t
            k1 = k0 + 1

            @pl.when(t > 0)
            def _():
                pltpu.make_async_copy(rows_b, out_sh.at[dst_v.at[0]],
                                      ssb).wait()

            pltpu.async_copy(hw_hbm.at[src_v.at[k1]], rows_b, gsb)
            pltpu.make_async_copy(hw_hbm.at[src_v.at[k0]], rows_a,
                                  gsa).wait()
            scale(rows_a, k0)
            pltpu.async_copy(rows_a, out_sh.at[dst_v.at[k0]], ssa, add=True)
            pltpu.make_async_copy(hw_hbm.at[src_v.at[k1]], rows_b,
                                  gsb).wait()
            scale(rows_b, k1)
            pltpu.async_copy(rows_b, out_sh.at[dst_v.at[k1]], ssb, add=True)

            @pl.when(t < npair - 1)
            def _():
                pltpu.make_async_copy(rows_a, out_sh.at[dst_v.at[0]],
                                      ssa).wait()
                pltpu.async_copy(hw_hbm.at[src_v.at[k0 + 2]], rows_a, gsa)

            return 0

        lax.fori_loop(0, npair, pair_body, 0)
        pltpu.make_async_copy(rows_a, out_sh.at[dst_v.at[0]], ssa).wait()
        pltpu.make_async_copy(rows_b, out_sh.at[dst_v.at[0]], ssb).wait()
        return 0

    lax.fori_loop(0, STAGES, st_body, 0)
    plsc.subcore_barrier()

    for k in range(OROW // CHUNK):
        pltpu.sync_copy(out_sh.at[pl.ds(obase + k * CHUNK, CHUNK)],
                        out_hbm.at[c, pl.ds(obase + k * CHUNK, CHUNK)])


@jax.jit
def kernel(x, edge_index, edge_attr, W, We, a_src, a_dst, a_edge,
           bias, gamma, beta):
    ecC, ecm8 = _p1(We, a_edge, edge_attr)
    ecT = ecC.T

    pad = EPAD - E
    src2 = jnp.pad(edge_index[0], (0, pad)).reshape(NW * ROWS_W, CHUNK)
    dst2 = jnp.pad(edge_index[1], (0, pad)).reshape(NW * ROWS_W, CHUNK)
    ec2 = jnp.pad(ecT, ((0, 0), (0, pad))).reshape(8, NW * ROWS_W, CHUNK)
    ecm = jnp.max(ecm8, axis=1)
    perm = jnp.argsort(edge_index[0])
    srcs2 = jnp.pad(edge_index[0][perm],
                    (0, pad)).reshape(NW * ROWS_W, CHUNK)
    dsts2 = jnp.pad(edge_index[1][perm],
                    (0, pad)).reshape(NW * ROWS_W, CHUNK)

    h = x
    for l in range(L):
        hw, s_col, d_col = _p2(h, W[l], a_src[l][None, :], a_dst[l][None, :])
        sd = jnp.zeros((2, NPAD), jnp.float32)
        sd = sd.at[0, :N].set(s_col[:, 0]).at[1, :N].set(d_col[:, 0])
        ecm_l = jnp.full((16,), ecm[l], jnp.float32)
        ex, den = _sc_attn(sd, ec2[l], ecm_l, src2, dst2)
        rec = _p2b(den)
        exs = jnp.pad(ex.reshape(-1)[:E][perm],
                      (0, pad)).reshape(NW * ROWS_W, CHUNK)
        pp = _sc_aggr(hw, srcs2, dsts2, exs, rec)
        h = _p3(pp, bias[l][None, :], gamma[l][None, :], beta[l][None, :])
    return h


# speedup vs baseline: 1.2745x; 1.2745x over previous
import functools

import jax
import jax.numpy as jnp
from jax import lax
from jax.experimental import pallas as pl
from jax.experimental.pallas import tpu as pltpu
from jax.experimental.pallas import tpu_sc as plsc

N = 10000
E = 320000
D = 128
L = 6

NC = 2
NS = 16
NW = NC * NS
NPAD = 10240
NT = NPAD // NS
EPAD = NW * NPAD
ROWS_W = 80
CHUNK = 128
OROW = NPAD // NS

_mesh = plsc.VectorSubcoreMesh(core_axis_name="c", subcore_axis_name="s")


def _dot_nt(a, b):
    return lax.dot_general(a, b, (((1,), (1,)), ((), ())),
                           preferred_element_type=jnp.float32)


_EB = 6400
_GB = E // _EB


def _p1_body(We_ref, ae_ref, ea_ref, ecC_ref, ecm_ref, acc_ref):
    i = pl.program_id(0)
    ecC_ref[...] = jnp.zeros((_EB, 8), jnp.float32)

    @pl.when(i == 0)
    def _():
        acc_ref[...] = jnp.zeros((8, D), jnp.float32)

    for l in range(L):
        he = jnp.dot(ea_ref[...], We_ref[l],
                     preferred_element_type=jnp.float32)
        ec = jnp.sum(he * ae_ref[l][None, :], axis=1, keepdims=True)
        ecC_ref[:, l:l + 1] = ec
        m = jnp.max(ec)
        acc_ref[l, :] = jnp.maximum(acc_ref[l, :], jnp.full((D,), m))

    @pl.when(i == _GB - 1)
    def _():
        ecm_ref[...] = acc_ref[...]


def _p1(We, a_edge, edge_attr):
    return pl.pallas_call(
        _p1_body,
        grid=(_GB,),
        in_specs=[
            pl.BlockSpec((L, D, D), lambda i: (0, 0, 0)),
            pl.BlockSpec((L, D), lambda i: (0, 0)),
            pl.BlockSpec((_EB, D), lambda i: (i, 0)),
        ],
        out_specs=[
            pl.BlockSpec((_EB, 8), lambda i: (i, 0)),
            pl.BlockSpec((8, D), lambda i: (0, 0)),
        ],
        out_shape=[
            jax.ShapeDtypeStruct((E, 8), jnp.float32),
            jax.ShapeDtypeStruct((8, D), jnp.float32),
        ],
        scratch_shapes=[pltpu.VMEM((8, D), jnp.float32)],
    )(We, a_edge, edge_attr)


def _p2_body(h_ref, w_ref, asrc_ref, adst_ref, hw_ref, s_ref, d_ref):
    h = h_ref[...]
    hw = jnp.dot(h, w_ref[...], preferred_element_type=jnp.float32)
    hw_ref[...] = hw
    s_ref[...] = jnp.sum(hw * asrc_ref[...], axis=1, keepdims=True)
    d_ref[...] = jnp.sum(hw * adst_ref[...], axis=1, keepdims=True)


def _p2(h, w_l, asrc_row, adst_row):
    return pl.pallas_call(
        _p2_body,
        out_shape=[
            jax.ShapeDtypeStruct((N, D), jnp.float32),
            jax.ShapeDtypeStruct((N, 1), jnp.float32),
            jax.ShapeDtypeStruct((N, 1), jnp.float32),
        ],
    )(h, w_l, asrc_row, adst_row)


def _bnprep_body(pp_ref, bias_ref, gamma_ref, beta_ref, w_ref, asrc_ref,
                 adst_ref, hw_ref, s_ref, d_ref):
    out = pp_ref[0, :N, :] + pp_ref[1, :N, :] + bias_ref[...]
    mu = jnp.mean(out, axis=0, keepdims=True)
    cen = out - mu
    var = jnp.mean(cen * cen, axis=0, keepdims=True)
    h = cen * lax.rsqrt(var + 1e-5) * gamma_ref[...] + beta_ref[...]
    hw = jnp.dot(h, w_ref[...], preferred_element_type=jnp.float32)
    hw_ref[...] = hw
    s_ref[...] = jnp.sum(hw * asrc_ref[...], axis=1, keepdims=True)
    d_ref[...] = jnp.sum(hw * adst_ref[...], axis=1, keepdims=True)


def _bnprep(pp, bias_row, gamma_row, beta_row, w_l, asrc_row, adst_row):
    return pl.pallas_call(
        _bnprep_body,
        out_shape=[
            jax.ShapeDtypeStruct((N, D), jnp.float32),
            jax.ShapeDtypeStruct((N, 1), jnp.float32),
            jax.ShapeDtypeStruct((N, 1), jnp.float32),
        ],
    )(pp, bias_row, gamma_row, beta_row, w_l, asrc_row, adst_row)


def _p3_body(pp_ref, bias_ref, gamma_ref, beta_ref, h_ref):
    out = pp_ref[0, :N, :] + pp_ref[1, :N, :] + bias_ref[...]
    mu = jnp.mean(out, axis=0, keepdims=True)
    cen = out - mu
    var = jnp.mean(cen * cen, axis=0, keepdims=True)
    h_ref[...] = cen * lax.rsqrt(var + 1e-5) * gamma_ref[...] + beta_ref[...]


def _p3(pp, bias_row, gamma_row, beta_row):
    return pl.pallas_call(
        _p3_body,
        out_shape=jax.ShapeDtypeStruct((N, D), jnp.float32),
    )(pp, bias_row, gamma_row, beta_row)


@functools.partial(
    pl.kernel,
    out_type=[
        jax.ShapeDtypeStruct((NW * ROWS_W, CHUNK), jnp.float32),
        jax.ShapeDtypeStruct((NC, NPAD), jnp.float32),
    ],
    mesh=_mesh,
    compiler_params=pltpu.CompilerParams(needs_layout_passes=False),
    scratch_types=[
        pltpu.VMEM((NPAD,), jnp.float32),
        pltpu.VMEM((NPAD,), jnp.float32),
        pltpu.VMEM((ROWS_W, CHUNK), jnp.int32),
        pltpu.VMEM((ROWS_W, CHUNK), jnp.int32),
        pltpu.VMEM((ROWS_W, CHUNK), jnp.float32),
        pltpu.VMEM((ROWS_W, CHUNK), jnp.float32),
        pltpu.VMEM((16,), jnp.float32),
        pltpu.VMEM((128,), jnp.float32),
        pltpu.VMEM((NT,), jnp.float32),
        pltpu.VMEM_SHARED((NPAD,), jnp.float32),
        pltpu.SemaphoreType.DMA,
    ],
)
def _sc_attn(sd_hbm, ec_hbm, ecm_hbm, src_hbm, dst_hbm,
             ex_hbm, den_hbm,
             s_v, d_v, src_v, dst_v, ec_v, ex_v, ecm_v, tmp_v, z_v, den_sh,
             dsem):
    c = lax.axis_index("c")
    sid = lax.axis_index("s")
    wid = c * NS + sid
    rbase = wid * ROWS_W

    pltpu.sync_copy(sd_hbm.at[0], s_v)
    pltpu.sync_copy(sd_hbm.at[1], d_v)
    pltpu.sync_copy(src_hbm.at[pl.ds(rbase, ROWS_W)], src_v)
    pltpu.sync_copy(dst_hbm.at[pl.ds(rbase, ROWS_W)], dst_v)
    pltpu.sync_copy(ec_hbm.at[pl.ds(rbase, ROWS_W)], ec_v)
    pltpu.sync_copy(ecm_hbm, ecm_v)

    def mx_body(j, carry):
        ms, md = carry
        ms = jnp.maximum(ms, s_v[pl.ds(j * 16, 16)])
        md = jnp.maximum(md, d_v[pl.ds(j * 16, 16)])
        return ms, md

    init = jnp.zeros((16,), jnp.float32)
    ms, md = lax.fori_loop(0, NPAD // 16, mx_body, (init, init))
    lanes = lax.iota(jnp.int32, 16)

    def lane_max(v):
        for sh in (8, 4, 2, 1):
            tmp_v[pl.ds(0, 16)] = v
            g = plsc.load_gather(tmp_v, [lanes ^ sh])
            v = jnp.maximum(v, g)
        return v

    cbound = lane_max(ms) + lane_max(md) + ecm_v[...]
    cvec = jnp.maximum(cbound, 0.0)

    def row_body(r, _):
        for q in range(CHUNK // 16):
            col = q * 16
            si = src_v[r, pl.ds(col, 16)]
            di = dst_v[r, pl.ds(col, 16)]
            sg = plsc.load_gather(s_v, [si])
            dg = plsc.load_gather(d_v, [di])
            lg = sg + dg + ec_v[r, pl.ds(col, 16)]
            lg = jnp.where(lg > 0, lg, lg * 0.2) - cvec
            ex = jnp.exp(lg)
            gid = wid * (ROWS_W * CHUNK) + r * CHUNK + col + lanes
            ex_v[r, pl.ds(col, 16)] = jnp.where(gid < E, ex, 0.0)
        return 0

    lax.fori_loop(0, ROWS_W, row_body, 0)

    pltpu.sync_copy(ex_v, ex_hbm.at[pl.ds(rbase, ROWS_W)])

    def z_body(j, _):
        z_v[pl.ds(j * 16, 16)] = jnp.zeros((16,), jnp.float32)
        return 0

    lax.fori_loop(0, NT // 16, z_body, 0)
    pltpu.sync_copy(z_v, den_sh.at[pl.ds(sid * NT, NT)])
    plsc.subcore_barrier()

    def sc_body(r, _):
        pltpu.async_copy(ex_v.at[r], den_sh.at[dst_v.at[r]], dsem, add=True)
        return 0

    lax.fori_loop(0, ROWS_W, sc_body, 0)

    def dr_body(r, _):
        pltpu.make_async_copy(ex_v.at[0], den_sh.at[dst_v.at[0]],
                              dsem).wait()
        return 0

    lax.fori_loop(0, ROWS_W, dr_body, 0)
    plsc.subcore_barrier()

    pltpu.sync_copy(den_sh.at[pl.ds(sid * NT, NT)],
                    den_hbm.at[c, pl.ds(sid * NT, NT)])


STAGES = 10
SROWS = ROWS_W // STAGES


@functools.partial(
    pl.kernel,
    out_type=jax.ShapeDtypeStruct((NC, NPAD, D), jnp.float32),
    mesh=_mesh,
    compiler_params=pltpu.CompilerParams(needs_layout_passes=False),
    scratch_types=[
        pltpu.VMEM((NPAD,), jnp.float32),
        pltpu.VMEM((SROWS, CHUNK), jnp.int32),
        pltpu.VMEM((SROWS, CHUNK), jnp.int32),
        pltpu.VMEM((SROWS, CHUNK), jnp.float32),
        pltpu.VMEM((CHUNK, D), jnp.float32),
        pltpu.VMEM((CHUNK, D), jnp.float32),
        pltpu.VMEM_SHARED((NPAD, D), jnp.float32),
        pltpu.VMEM((NT,), jnp.float32),
        pltpu.VMEM((NT,), jnp.float32),
        pltpu.VMEM_SHARED((NPAD,), jnp.float32),
        pltpu.SemaphoreType.DMA,
        pltpu.SemaphoreType.DMA,
        pltpu.SemaphoreType.DMA,
        pltpu.SemaphoreType.DMA,
    ],
)
def _sc_aggr(hw_hbm, src_hbm, dst_hbm, ex_hbm, den_hbm,
             out_hbm,
             rec_v, src_v, dst_v, ex_v, rows_a, rows_b, out_sh,
             den_a, den_b, rec_sh,
             gsa, gsb, ssa, ssb):
    c = lax.axis_index("c")
    sid = lax.axis_index("s")
    wid = c * NS + sid
    rbase = wid * ROWS_W

    tslice = pl.ds(sid * NT, NT)
    pltpu.sync_copy(den_hbm.at[0, tslice], den_a)
    pltpu.sync_copy(den_hbm.at[1, tslice], den_b)

    def rc_body(j, _):
        sl = pl.ds(j * 16, 16)
        den_a[sl] = 1.0 / (den_a[sl] + den_b[sl] + 1e-16)
        return 0

    lax.fori_loop(0, NT // 16, rc_body, 0)
    pltpu.sync_copy(den_a, rec_sh.at[tslice])
    plsc.subcore_barrier()
    pltpu.sync_copy(rec_sh, rec_v)

    def zr_body(r, _):
        def zq_body(q, _):
            rows_a[r, pl.ds(q * 16, 16)] = jnp.zeros((16,), jnp.float32)
            return 0

        lax.fori_loop(0, D // 16, zq_body, 0)
        return 0

    lax.fori_loop(0, CHUNK, zr_body, 0)
    obase = sid * OROW
    for k in range(OROW // CHUNK):
        pltpu.sync_copy(rows_a, out_sh.at[pl.ds(obase + k * CHUNK, CHUNK)])
    plsc.subcore_barrier()

    def scale(rows_ref, k):
        def r_body(r, _):
            a16 = plsc.load_gather(
                ex_v, [jnp.full((16,), k, jnp.int32),
                       jnp.full((16,), r, jnp.int32)])
            for b in range(D // 16):
                sl = pl.ds(b * 16, 16)
                rows_ref[r, sl] = rows_ref[r, sl] * a16
            return 0

        lax.fori_loop(0, CHUNK, r_body, 0)

    def st_body(st, _):
        sbase = rbase + st * SROWS
        pltpu.sync_copy(src_hbm.at[pl.ds(sbase, SROWS)], src_v)
        pltpu.sync_copy(dst_hbm.at[pl.ds(sbase, SROWS)], dst_v)
        pltpu.sync_copy(ex_hbm.at[pl.ds(sbase, SROWS)], ex_v)

        def al_body(r, _):
            for q in range(CHUNK // 16):
                sl = pl.ds(q * 16, 16)
                di = dst_v[r, sl]
                ex_v[r, sl] = ex_v[r, sl] * plsc.load_gather(rec_v, [di])
            return 0

        lax.fori_loop(0, SROWS, al_body, 0)

        pltpu.async_copy(hw_hbm.at[src_v.at[0]], rows_a, gsa)

        npair = SROWS // 2

        def pair_body(t, _):
            k0 = 2 * t
            k1 = k0 + 1

            @pl.when(t > 0)
            def _():
                pltpu.make_async_copy(rows_b, out_sh.at[dst_v.at[0]],
                                      ssb).wait()

            pltpu.async_copy(hw_hbm.at[src_v.at[k1]], rows_b, gsb)
            pltpu.make_async_copy(hw_hbm.at[src_v.at[k0]], rows_a,
                                  gsa).wait()
            scale(rows_a, k0)
            pltpu.async_copy(rows_a, out_sh.at[dst_v.at[k0]], ssa, add=True)
            pltpu.make_async_copy(hw_hbm.at[src_v.at[k1]], rows_b,
                                  gsb).wait()
            scale(rows_b, k1)
            pltpu.async_copy(rows_b, out_sh.at[dst_v.at[k1]], ssb, add=True)

            @pl.when(t < npair - 1)
            def _():
                pltpu.make_async_copy(rows_a, out_sh.at[dst_v.at[0]],
                                      ssa).wait()
                pltpu.async_copy(hw_hbm.at[src_v.at[k0 + 2]], rows_a, gsa)

            return 0

        lax.fori_loop(0, npair, pair_body, 0)
        pltpu.make_async_copy(rows_a, out_sh.at[dst_v.at[0]], ssa).wait()
        pltpu.make_async_copy(rows_b, out_sh.at[dst_v.at[0]], ssb).wait()
        return 0

    lax.fori_loop(0, STAGES, st_body, 0)
    plsc.subcore_barrier()

    for k in range(OROW // CHUNK):
        pltpu.sync_copy(out_sh.at[pl.ds(obase + k * CHUNK, CHUNK)],
                        out_hbm.at[c, pl.ds(obase + k * CHUNK, CHUNK)])


@jax.jit
def kernel(x, edge_index, edge_attr, W, We, a_src, a_dst, a_edge,
           bias, gamma, beta):
    ecC, ecm8 = _p1(We, a_edge, edge_attr)
    ecT = ecC.T

    pad = EPAD - E
    src2 = jnp.pad(edge_index[0], (0, pad)).reshape(NW * ROWS_W, CHUNK)
    dst2 = jnp.pad(edge_index[1], (0, pad)).reshape(NW * ROWS_W, CHUNK)
    ec2 = jnp.pad(ecT, ((0, 0), (0, pad))).reshape(8, NW * ROWS_W, CHUNK)
    ecm = jnp.max(ecm8, axis=1)

    hw, s_col, d_col = _p2(x, W[0], a_src[0][None, :], a_dst[0][None, :])
    for l in range(L):
        sd = jnp.zeros((2, NPAD), jnp.float32)
        sd = sd.at[0, :N].set(s_col[:, 0]).at[1, :N].set(d_col[:, 0])
        ecm_l = jnp.full((16,), ecm[l], jnp.float32)
        ex, den = _sc_attn(sd, ec2[l], ecm_l, src2, dst2)
        pp = _sc_aggr(hw, src2, dst2, ex, den)
        if l < L - 1:
            hw, s_col, d_col = _bnprep(
                pp, bias[l][None, :], gamma[l][None, :], beta[l][None, :],
                W[l + 1], a_src[l + 1][None, :], a_dst[l + 1][None, :])
        else:
            h = _p3(pp, bias[l][None, :], gamma[l][None, :],
                    beta[l][None, :])
    return h
